# Initial kernel scaffold; baseline (speedup 1.0000x reference)
import jax
import jax.numpy as jnp
from jax.experimental import pallas as pl


def kernel(edge_index, features, W1, b1, g1, beta1, W2, b2, g2, beta2, W3, b3):
    raise NotImplementedError("write your pallas kernel here")


# speedup vs baseline: 18.0378x; 18.0378x over previous
import functools

import jax
import jax.numpy as jnp
from jax import lax
from jax.experimental import pallas as pl
from jax.experimental.pallas import tpu as pltpu
from jax.experimental.pallas import tpu_sc as plsc

N = 10000
E = 320000
D_IN = 128
H = 16
D_OUT = 128

NC = 2
NS = 16
NW = NC * NS
CHUNK = 128
CHUNKS_PW = 80
EPW = CHUNK * CHUNKS_PW
E_PAD = EPW * NW
N_ACC = 10240
ROWS_PT = N_ACC // NS

def _sc_degree_body(row_hbm, ones_hbm, zeros_hbm, out_hbm,
                    rows_v, msg_v, acc_sh, sem):
    cid = lax.axis_index("c")
    sid = lax.axis_index("s")
    wid = cid * NS + sid
    pltpu.sync_copy(zeros_hbm.at[pl.ds(sid * ROWS_PT, ROWS_PT)],
                    acc_sh.at[pl.ds(sid * ROWS_PT, ROWS_PT)])
    pltpu.sync_copy(row_hbm.at[wid], rows_v)
    pltpu.sync_copy(ones_hbm, msg_v)
    plsc.subcore_barrier()

    def body(c, carry):
        pltpu.sync_copy(msg_v, acc_sh.at[rows_v.at[c]], add=True)
        return carry
    lax.fori_loop(0, CHUNKS_PW, body, 0)
    plsc.subcore_barrier()
    pltpu.sync_copy(acc_sh.at[pl.ds(sid * ROWS_PT, ROWS_PT)],
                    out_hbm.at[cid, pl.ds(sid * ROWS_PT, ROWS_PT)])


@functools.cache
def _sc_degree():
    mesh = plsc.VectorSubcoreMesh(
        core_axis_name="c", subcore_axis_name="s",
        num_cores=NC, num_subcores=NS)
    return pl.kernel(
        _sc_degree_body, mesh=mesh,
        out_type=jax.ShapeDtypeStruct((NC, N_ACC, H), jnp.float32),
        scratch_types=[
            pltpu.VMEM((CHUNKS_PW, CHUNK), jnp.int32),
            pltpu.VMEM((CHUNK, H), jnp.float32),
            pltpu.VMEM_SHARED((N_ACC, H), jnp.float32),
            pltpu.SemaphoreType.DMA,
        ],
        compiler_params=pltpu.CompilerParams(use_tc_tiling_on_sc=False),
        name="sc_degree",
    )


def _sc_agg_body(row_hbm, col_hbm, y_hbm, zeros_hbm, out_hbm,
                 rows_v, cols_v, msg_v, acc_sh, sem):
    cid = lax.axis_index("c")
    sid = lax.axis_index("s")
    wid = cid * NS + sid
    pltpu.sync_copy(zeros_hbm.at[pl.ds(sid * ROWS_PT, ROWS_PT)],
                    acc_sh.at[pl.ds(sid * ROWS_PT, ROWS_PT)])
    pltpu.sync_copy(row_hbm.at[wid], rows_v)
    pltpu.sync_copy(col_hbm.at[wid], cols_v)
    plsc.subcore_barrier()

    def body(c, carry):
        pltpu.async_copy(y_hbm.at[cols_v.at[c]], msg_v, sem).wait()
        pltpu.sync_copy(msg_v, acc_sh.at[rows_v.at[c]], add=True)
        return carry
    lax.fori_loop(0, CHUNKS_PW, body, 0)
    plsc.subcore_barrier()
    pltpu.sync_copy(acc_sh.at[pl.ds(sid * ROWS_PT, ROWS_PT)],
                    out_hbm.at[cid, pl.ds(sid * ROWS_PT, ROWS_PT)])


@functools.cache
def _sc_agg():
    mesh = plsc.VectorSubcoreMesh(
        core_axis_name="c", subcore_axis_name="s",
        num_cores=NC, num_subcores=NS)
    return pl.kernel(
        _sc_agg_body, mesh=mesh,
        out_type=jax.ShapeDtypeStruct((NC, N_ACC, H), jnp.float32),
        scratch_types=[
            pltpu.VMEM((CHUNKS_PW, CHUNK), jnp.int32),
            pltpu.VMEM((CHUNKS_PW, CHUNK), jnp.int32),
            pltpu.VMEM((CHUNK, H), jnp.float32),
            pltpu.VMEM_SHARED((N_ACC, H), jnp.float32),
            pltpu.SemaphoreType.DMA,
        ],
        compiler_params=pltpu.CompilerParams(use_tc_tiling_on_sc=False),
        name="sc_agg",
    )


def _tc1_body(dega, degb, x, w1t, w1b, dinv_o, u1_o, p1_o):
    deg = dega[...] + degb[...] + 1.0
    dinv = lax.rsqrt(deg)
    z = jnp.dot(x[...], w1b[...], preferred_element_type=jnp.float32)
    dinv_o[...] = dinv
    u1_o[...] = dinv * z
    p1_o[...] = jnp.dot(x[...], w1t[...], preferred_element_type=jnp.float32)


def _bn_relu(h, g, beta):
    mu = jnp.mean(h, axis=0, keepdims=True)
    var = jnp.mean((h - mu) * (h - mu), axis=0, keepdims=True)
    return jnp.maximum((h - mu) / jnp.sqrt(var + 1e-5) * g + beta, 0.0)


def _tc_after1_body(sa, sb, u, p, dinv_r, b, g, beta, wt_next,
                    vn_o, pn_o):
    dinv = dinv_r[...]
    h = p[...] + dinv * (sa[...] + sb[...] + u[...]) + b[...]
    hn = _bn_relu(h, g[...], beta[...])
    vn_o[...] = dinv * hn
    pn_o[...] = jnp.dot(hn, wt_next[...], preferred_element_type=jnp.float32)


def _tc_mid_body(ta, tb, v, p, dinv_r, b, g, beta, wb_prev, wt_next,
                 vn_o, pn_o):
    dinv = dinv_r[...]
    agg = dinv * (ta[...] + tb[...] + v[...])
    h = (p[...] +
         jnp.dot(agg, wb_prev[...], preferred_element_type=jnp.float32) +
         b[...])
    hn = _bn_relu(h, g[...], beta[...])
    vn_o[...] = dinv * hn
    pn_o[...] = jnp.dot(hn, wt_next[...], preferred_element_type=jnp.float32)


def _tc_out_body(sa, sb, u, p, dinv_r, wb, b, out_o):
    agg = dinv_r[...] * (sa[...] + sb[...] + u[...])
    out_o[...] = (p[...] +
                  jnp.dot(agg, wb[...], preferred_element_type=jnp.float32) +
                  b[...])


def kernel(edge_index, features, W1, b1, g1, beta1, W2, b2, g2, beta2, W3, b3):
    f32 = jnp.float32
    row = edge_index[0]
    col = edge_index[1]
    pad = E_PAD - E
    row_p = jnp.concatenate([row, jnp.full((pad,), N, jnp.int32)])
    col_p = jnp.concatenate([col, jnp.zeros((pad,), jnp.int32)])
    row_p = row_p.reshape(NW, CHUNKS_PW, CHUNK)
    col_p = col_p.reshape(NW, CHUNKS_PW, CHUNK)
    zeros_acc = jnp.zeros((N_ACC, H), f32)
    ones_msg = jnp.ones((CHUNK, H), f32)

    w1t, w1b = W1[:D_IN], W1[D_IN:]
    w2t, w2b = W2[:H], W2[H:]
    w3t, w3b = W3[:H], W3[H:]
    b1r, g1r, bt1 = b1.reshape(1, H), g1.reshape(1, H), beta1.reshape(1, H)
    b2r, g2r, bt2 = b2.reshape(1, H), g2.reshape(1, H), beta2.reshape(1, H)
    b3r = b3.reshape(1, D_OUT)

    deg2 = _sc_degree()(row_p, ones_msg, zeros_acc)

    dinv, u1, p1 = pl.pallas_call(
        _tc1_body,
        out_shape=[jax.ShapeDtypeStruct((N, H), f32)] * 3,
    )(deg2[0, :N], deg2[1, :N], features, w1t, w1b)

    s1 = _sc_agg()(row_p, col_p, u1, zeros_acc)
    v2, p2 = pl.pallas_call(
        _tc_after1_body,
        out_shape=[jax.ShapeDtypeStruct((N, H), f32)] * 2,
    )(s1[0, :N], s1[1, :N], u1, p1, dinv, b1r, g1r, bt1, w2t)

    t2 = _sc_agg()(row_p, col_p, v2, zeros_acc)
    v3, p3 = pl.pallas_call(
        _tc_mid_body,
        out_shape=[jax.ShapeDtypeStruct((N, H), f32),
                   jax.ShapeDtypeStruct((N, D_OUT), f32)],
    )(t2[0, :N], t2[1, :N], v2, p2, dinv, b2r, g2r, bt2, w2b, w3t)

    t3 = _sc_agg()(row_p, col_p, v3, zeros_acc)
    out = pl.pallas_call(
        _tc_out_body,
        out_shape=jax.ShapeDtypeStruct((N, D_OUT), f32),
    )(t3[0, :N], t3[1, :N], v3, p3, dinv, w3b, b3r)
    return out


# speedup vs baseline: 24.2778x; 1.3459x over previous
import functools

import jax
import jax.numpy as jnp
from jax import lax
from jax.experimental import pallas as pl
from jax.experimental.pallas import tpu as pltpu
from jax.experimental.pallas import tpu_sc as plsc

N = 10000
E = 320000
D_IN = 128
H = 16
D_OUT = 128

NC = 2
NS = 16
NW = NC * NS
CHUNK = 128
CHUNKS_PW = 80
EPW = CHUNK * CHUNKS_PW
E_PAD = EPW * NW
N_ACC = 10240
ROWS_PT = N_ACC // NS

def _sc_degree_body(row_hbm, ones_hbm, zeros_hbm, out_hbm,
                    rows_v, msg_v, acc_sh, sem):
    cid = lax.axis_index("c")
    sid = lax.axis_index("s")
    wid = cid * NS + sid
    pltpu.sync_copy(zeros_hbm.at[pl.ds(sid * ROWS_PT, ROWS_PT)],
                    acc_sh.at[pl.ds(sid * ROWS_PT, ROWS_PT)])
    pltpu.sync_copy(row_hbm.at[wid], rows_v)
    pltpu.sync_copy(ones_hbm, msg_v)
    plsc.subcore_barrier()

    def body(c, carry):
        pltpu.sync_copy(msg_v, acc_sh.at[rows_v.at[c]], add=True)
        return carry
    lax.fori_loop(0, CHUNKS_PW, body, 0)
    plsc.subcore_barrier()
    pltpu.sync_copy(acc_sh.at[pl.ds(sid * ROWS_PT, ROWS_PT)],
                    out_hbm.at[cid, pl.ds(sid * ROWS_PT, ROWS_PT)])


@functools.cache
def _sc_degree():
    mesh = plsc.VectorSubcoreMesh(
        core_axis_name="c", subcore_axis_name="s",
        num_cores=NC, num_subcores=NS)
    return pl.kernel(
        _sc_degree_body, mesh=mesh,
        out_type=jax.ShapeDtypeStruct((NC, N_ACC, H), jnp.float32),
        scratch_types=[
            pltpu.VMEM((CHUNKS_PW, CHUNK), jnp.int32),
            pltpu.VMEM((CHUNK, H), jnp.float32),
            pltpu.VMEM_SHARED((N_ACC, H), jnp.float32),
            pltpu.SemaphoreType.DMA,
        ],
        compiler_params=pltpu.CompilerParams(use_tc_tiling_on_sc=False),
        name="sc_degree",
    )


NBUF = 4
BLKS = CHUNKS_PW // NBUF


def _sc_agg_body(row_hbm, col_hbm, y_hbm, zeros_hbm, out_hbm,
                 rows_v, cols_v, msg_v, acc_sh, sem0, sem1, sem2, sem3):
    sems = (sem0, sem1, sem2, sem3)
    cid = lax.axis_index("c")
    sid = lax.axis_index("s")
    wid = cid * NS + sid
    pltpu.sync_copy(zeros_hbm.at[pl.ds(sid * ROWS_PT, ROWS_PT)],
                    acc_sh.at[pl.ds(sid * ROWS_PT, ROWS_PT)])
    pltpu.sync_copy(row_hbm.at[wid], rows_v)
    pltpu.sync_copy(col_hbm.at[wid], cols_v)
    plsc.subcore_barrier()

    for b in range(NBUF):
        pltpu.async_copy(y_hbm.at[cols_v.at[b]], msg_v.at[b], sems[b])

    def blk(cb, carry):
        for b in range(NBUF):
            c = cb * NBUF + b
            pltpu.make_async_copy(y_hbm.at[cols_v.at[c]],
                                  msg_v.at[b], sems[b]).wait()
            pltpu.sync_copy(msg_v.at[b], acc_sh.at[rows_v.at[c]], add=True)
            pltpu.async_copy(y_hbm.at[cols_v.at[c + NBUF]],
                             msg_v.at[b], sems[b])
        return carry
    lax.fori_loop(0, BLKS - 1, blk, 0)
    for b in range(NBUF):
        c = (BLKS - 1) * NBUF + b
        pltpu.make_async_copy(y_hbm.at[cols_v.at[c]],
                              msg_v.at[b], sems[b]).wait()
        pltpu.sync_copy(msg_v.at[b], acc_sh.at[rows_v.at[c]], add=True)

    plsc.subcore_barrier()
    pltpu.sync_copy(acc_sh.at[pl.ds(sid * ROWS_PT, ROWS_PT)],
                    out_hbm.at[cid, pl.ds(sid * ROWS_PT, ROWS_PT)])


@functools.cache
def _sc_agg():
    mesh = plsc.VectorSubcoreMesh(
        core_axis_name="c", subcore_axis_name="s",
        num_cores=NC, num_subcores=NS)
    return pl.kernel(
        _sc_agg_body, mesh=mesh,
        out_type=jax.ShapeDtypeStruct((NC, N_ACC, H), jnp.float32),
        scratch_types=[
            pltpu.VMEM((CHUNKS_PW, CHUNK), jnp.int32),
            pltpu.VMEM((CHUNKS_PW, CHUNK), jnp.int32),
            pltpu.VMEM((NBUF, CHUNK, H), jnp.float32),
            pltpu.VMEM_SHARED((N_ACC, H), jnp.float32),
            pltpu.SemaphoreType.DMA,
            pltpu.SemaphoreType.DMA,
            pltpu.SemaphoreType.DMA,
            pltpu.SemaphoreType.DMA,
        ],
        compiler_params=pltpu.CompilerParams(use_tc_tiling_on_sc=False),
        name="sc_agg",
    )


def _tc1_body(dega, degb, x, w1t, w1b, dinv_o, u1_o, p1_o):
    deg = dega[...] + degb[...] + 1.0
    dinv = lax.rsqrt(deg)
    z = jnp.dot(x[...], w1b[...], preferred_element_type=jnp.float32)
    dinv_o[...] = dinv
    u1_o[...] = dinv * z
    p1_o[...] = jnp.dot(x[...], w1t[...], preferred_element_type=jnp.float32)


def _bn_relu(h, g, beta):
    mu = jnp.mean(h, axis=0, keepdims=True)
    var = jnp.mean((h - mu) * (h - mu), axis=0, keepdims=True)
    return jnp.maximum((h - mu) / jnp.sqrt(var + 1e-5) * g + beta, 0.0)


def _tc_after1_body(sa, sb, u, p, dinv_r, b, g, beta, wt_next,
                    vn_o, pn_o):
    dinv = dinv_r[...]
    h = p[...] + dinv * (sa[...] + sb[...] + u[...]) + b[...]
    hn = _bn_relu(h, g[...], beta[...])
    vn_o[...] = dinv * hn
    pn_o[...] = jnp.dot(hn, wt_next[...], preferred_element_type=jnp.float32)


def _tc_mid_body(ta, tb, v, p, dinv_r, b, g, beta, wb_prev, wt_next,
                 vn_o, pn_o):
    dinv = dinv_r[...]
    agg = dinv * (ta[...] + tb[...] + v[...])
    h = (p[...] +
         jnp.dot(agg, wb_prev[...], preferred_element_type=jnp.float32) +
         b[...])
    hn = _bn_relu(h, g[...], beta[...])
    vn_o[...] = dinv * hn
    pn_o[...] = jnp.dot(hn, wt_next[...], preferred_element_type=jnp.float32)


def _tc_out_body(sa, sb, u, p, dinv_r, wb, b, out_o):
    agg = dinv_r[...] * (sa[...] + sb[...] + u[...])
    out_o[...] = (p[...] +
                  jnp.dot(agg, wb[...], preferred_element_type=jnp.float32) +
                  b[...])


def kernel(edge_index, features, W1, b1, g1, beta1, W2, b2, g2, beta2, W3, b3):
    f32 = jnp.float32
    row = edge_index[0]
    col = edge_index[1]
    pad = E_PAD - E
    row_p = jnp.concatenate([row, jnp.full((pad,), N, jnp.int32)])
    col_p = jnp.concatenate([col, jnp.zeros((pad,), jnp.int32)])
    row_p = row_p.reshape(NW, CHUNKS_PW, CHUNK)
    col_p = col_p.reshape(NW, CHUNKS_PW, CHUNK)
    zeros_acc = jnp.zeros((N_ACC, H), f32)
    ones_msg = jnp.ones((CHUNK, H), f32)

    w1t, w1b = W1[:D_IN], W1[D_IN:]
    w2t, w2b = W2[:H], W2[H:]
    w3t, w3b = W3[:H], W3[H:]
    b1r, g1r, bt1 = b1.reshape(1, H), g1.reshape(1, H), beta1.reshape(1, H)
    b2r, g2r, bt2 = b2.reshape(1, H), g2.reshape(1, H), beta2.reshape(1, H)
    b3r = b3.reshape(1, D_OUT)

    deg2 = _sc_degree()(row_p, ones_msg, zeros_acc)

    dinv, u1, p1 = pl.pallas_call(
        _tc1_body,
        out_shape=[jax.ShapeDtypeStruct((N, H), f32)] * 3,
    )(deg2[0, :N], deg2[1, :N], features, w1t, w1b)

    s1 = _sc_agg()(row_p, col_p, u1, zeros_acc)
    v2, p2 = pl.pallas_call(
        _tc_after1_body,
        out_shape=[jax.ShapeDtypeStruct((N, H), f32)] * 2,
    )(s1[0, :N], s1[1, :N], u1, p1, dinv, b1r, g1r, bt1, w2t)

    t2 = _sc_agg()(row_p, col_p, v2, zeros_acc)
    v3, p3 = pl.pallas_call(
        _tc_mid_body,
        out_shape=[jax.ShapeDtypeStruct((N, H), f32),
                   jax.ShapeDtypeStruct((N, D_OUT), f32)],
    )(t2[0, :N], t2[1, :N], v2, p2, dinv, b2r, g2r, bt2, w2b, w3t)

    t3 = _sc_agg()(row_p, col_p, v3, zeros_acc)
    out = pl.pallas_call(
        _tc_out_body,
        out_shape=jax.ShapeDtypeStruct((N, D_OUT), f32),
    )(t3[0, :N], t3[1, :N], v3, p3, dinv, w3b, b3r)
    return out


# speedup vs baseline: 34.4519x; 1.4191x over previous
import functools

import jax
import jax.numpy as jnp
from jax import lax
from jax.experimental import pallas as pl
from jax.experimental.pallas import tpu as pltpu
from jax.experimental.pallas import tpu_sc as plsc

N = 10000
E = 320000
D_IN = 128
H = 16
D_OUT = 128

NC = 2
NS = 16
NW = NC * NS
CHUNK = 128
CHUNKS_PW = 80
EPW = CHUNK * CHUNKS_PW
E_PAD = EPW * NW
N_ACC = 10240
ROWS_PT = N_ACC // NS

def _sc_degree_body(row_hbm, ones_hbm, zeros_hbm, out_hbm,
                    rows_v, msg_v, acc_sh, sem):
    cid = lax.axis_index("c")
    sid = lax.axis_index("s")
    wid = cid * NS + sid
    pltpu.sync_copy(zeros_hbm.at[pl.ds(sid * ROWS_PT, ROWS_PT)],
                    acc_sh.at[pl.ds(sid * ROWS_PT, ROWS_PT)])
    pltpu.sync_copy(row_hbm.at[wid], rows_v)
    pltpu.sync_copy(ones_hbm, msg_v)
    plsc.subcore_barrier()

    def body(c, carry):
        pltpu.sync_copy(msg_v, acc_sh.at[rows_v.at[c]], add=True)
        return carry
    lax.fori_loop(0, CHUNKS_PW, body, 0)
    plsc.subcore_barrier()
    pltpu.sync_copy(acc_sh.at[pl.ds(sid * ROWS_PT, ROWS_PT)],
                    out_hbm.at[cid, pl.ds(sid * ROWS_PT, ROWS_PT)])


@functools.cache
def _sc_degree():
    mesh = plsc.VectorSubcoreMesh(
        core_axis_name="c", subcore_axis_name="s",
        num_cores=NC, num_subcores=NS)
    return pl.kernel(
        _sc_degree_body, mesh=mesh,
        out_type=jax.ShapeDtypeStruct((NC, N_ACC, H), jnp.float32),
        scratch_types=[
            pltpu.VMEM((CHUNKS_PW, CHUNK), jnp.int32),
            pltpu.VMEM((CHUNK, H), jnp.float32),
            pltpu.VMEM_SHARED((N_ACC, H), jnp.float32),
            pltpu.SemaphoreType.DMA,
        ],
        compiler_params=pltpu.CompilerParams(use_tc_tiling_on_sc=False),
        name="sc_degree",
    )


NBUF = 4
BLKS = CHUNKS_PW // NBUF


N_Y = N // NS


def _sc_agg_body(row_hbm, col_hbm, y_hbm, zeros_hbm, out_hbm,
                 rows_v, cols_v, msg_v, acc_sh, y_sh, sem0, sem1, sem2, sem3):
    sems = (sem0, sem1, sem2, sem3)
    cid = lax.axis_index("c")
    sid = lax.axis_index("s")
    wid = cid * NS + sid
    pltpu.sync_copy(zeros_hbm.at[pl.ds(sid * ROWS_PT, ROWS_PT)],
                    acc_sh.at[pl.ds(sid * ROWS_PT, ROWS_PT)])
    pltpu.sync_copy(y_hbm.at[pl.ds(sid * N_Y, N_Y)],
                    y_sh.at[pl.ds(sid * N_Y, N_Y)])
    pltpu.sync_copy(row_hbm.at[wid], rows_v)
    pltpu.sync_copy(col_hbm.at[wid], cols_v)
    plsc.subcore_barrier()

    for b in range(NBUF):
        pltpu.async_copy(y_sh.at[cols_v.at[b]], msg_v.at[b], sems[b])

    def blk(cb, carry):
        for b in range(NBUF):
            c = cb * NBUF + b
            pltpu.make_async_copy(y_sh.at[cols_v.at[c]],
                                  msg_v.at[b], sems[b]).wait()
            pltpu.sync_copy(msg_v.at[b], acc_sh.at[rows_v.at[c]], add=True)
            pltpu.async_copy(y_sh.at[cols_v.at[c + NBUF]],
                             msg_v.at[b], sems[b])
        return carry
    lax.fori_loop(0, BLKS - 1, blk, 0)
    for b in range(NBUF):
        c = (BLKS - 1) * NBUF + b
        pltpu.make_async_copy(y_sh.at[cols_v.at[c]],
                              msg_v.at[b], sems[b]).wait()
        pltpu.sync_copy(msg_v.at[b], acc_sh.at[rows_v.at[c]], add=True)

    plsc.subcore_barrier()
    pltpu.sync_copy(acc_sh.at[pl.ds(sid * ROWS_PT, ROWS_PT)],
                    out_hbm.at[cid, pl.ds(sid * ROWS_PT, ROWS_PT)])


@functools.cache
def _sc_agg():
    mesh = plsc.VectorSubcoreMesh(
        core_axis_name="c", subcore_axis_name="s",
        num_cores=NC, num_subcores=NS)
    return pl.kernel(
        _sc_agg_body, mesh=mesh,
        out_type=jax.ShapeDtypeStruct((NC, N_ACC, H), jnp.float32),
        scratch_types=[
            pltpu.VMEM((CHUNKS_PW, CHUNK), jnp.int32),
            pltpu.VMEM((CHUNKS_PW, CHUNK), jnp.int32),
            pltpu.VMEM((NBUF, CHUNK, H), jnp.float32),
            pltpu.VMEM_SHARED((N_ACC, H), jnp.float32),
            pltpu.VMEM_SHARED((N, H), jnp.float32),
            pltpu.SemaphoreType.DMA,
            pltpu.SemaphoreType.DMA,
            pltpu.SemaphoreType.DMA,
            pltpu.SemaphoreType.DMA,
        ],
        compiler_params=pltpu.CompilerParams(use_tc_tiling_on_sc=False),
        name="sc_agg",
    )


def _tc1_body(dega, degb, x, w1t, w1b, dinv_o, u1_o, p1_o):
    deg = dega[...] + degb[...] + 1.0
    dinv = lax.rsqrt(deg)
    z = jnp.dot(x[...], w1b[...], preferred_element_type=jnp.float32)
    dinv_o[...] = dinv
    u1_o[...] = dinv * z
    p1_o[...] = jnp.dot(x[...], w1t[...], preferred_element_type=jnp.float32)


def _bn_relu(h, g, beta):
    mu = jnp.mean(h, axis=0, keepdims=True)
    var = jnp.mean((h - mu) * (h - mu), axis=0, keepdims=True)
    return jnp.maximum((h - mu) / jnp.sqrt(var + 1e-5) * g + beta, 0.0)


def _tc_after1_body(sa, sb, u, p, dinv_r, b, g, beta, wt_next,
                    vn_o, pn_o):
    dinv = dinv_r[...]
    h = p[...] + dinv * (sa[...] + sb[...] + u[...]) + b[...]
    hn = _bn_relu(h, g[...], beta[...])
    vn_o[...] = dinv * hn
    pn_o[...] = jnp.dot(hn, wt_next[...], preferred_element_type=jnp.float32)


def _tc_mid_body(ta, tb, v, p, dinv_r, b, g, beta, wb_prev, wt_next,
                 vn_o, pn_o):
    dinv = dinv_r[...]
    agg = dinv * (ta[...] + tb[...] + v[...])
    h = (p[...] +
         jnp.dot(agg, wb_prev[...], preferred_element_type=jnp.float32) +
         b[...])
    hn = _bn_relu(h, g[...], beta[...])
    vn_o[...] = dinv * hn
    pn_o[...] = jnp.dot(hn, wt_next[...], preferred_element_type=jnp.float32)


def _tc_out_body(sa, sb, u, p, dinv_r, wb, b, out_o):
    agg = dinv_r[...] * (sa[...] + sb[...] + u[...])
    out_o[...] = (p[...] +
                  jnp.dot(agg, wb[...], preferred_element_type=jnp.float32) +
                  b[...])


def kernel(edge_index, features, W1, b1, g1, beta1, W2, b2, g2, beta2, W3, b3):
    f32 = jnp.float32
    row = edge_index[0]
    col = edge_index[1]
    pad = E_PAD - E
    row_p = jnp.concatenate([row, jnp.full((pad,), N, jnp.int32)])
    col_p = jnp.concatenate([col, jnp.zeros((pad,), jnp.int32)])
    row_p = row_p.reshape(NW, CHUNKS_PW, CHUNK)
    col_p = col_p.reshape(NW, CHUNKS_PW, CHUNK)
    zeros_acc = jnp.zeros((N_ACC, H), f32)
    ones_msg = jnp.ones((CHUNK, H), f32)

    w1t, w1b = W1[:D_IN], W1[D_IN:]
    w2t, w2b = W2[:H], W2[H:]
    w3t, w3b = W3[:H], W3[H:]
    b1r, g1r, bt1 = b1.reshape(1, H), g1.reshape(1, H), beta1.reshape(1, H)
    b2r, g2r, bt2 = b2.reshape(1, H), g2.reshape(1, H), beta2.reshape(1, H)
    b3r = b3.reshape(1, D_OUT)

    deg2 = _sc_degree()(row_p, ones_msg, zeros_acc)

    dinv, u1, p1 = pl.pallas_call(
        _tc1_body,
        out_shape=[jax.ShapeDtypeStruct((N, H), f32)] * 3,
    )(deg2[0, :N], deg2[1, :N], features, w1t, w1b)

    s1 = _sc_agg()(row_p, col_p, u1, zeros_acc)
    v2, p2 = pl.pallas_call(
        _tc_after1_body,
        out_shape=[jax.ShapeDtypeStruct((N, H), f32)] * 2,
    )(s1[0, :N], s1[1, :N], u1, p1, dinv, b1r, g1r, bt1, w2t)

    t2 = _sc_agg()(row_p, col_p, v2, zeros_acc)
    v3, p3 = pl.pallas_call(
        _tc_mid_body,
        out_shape=[jax.ShapeDtypeStruct((N, H), f32),
                   jax.ShapeDtypeStruct((N, D_OUT), f32)],
    )(t2[0, :N], t2[1, :N], v2, p2, dinv, b2r, g2r, bt2, w2b, w3t)

    t3 = _sc_agg()(row_p, col_p, v3, zeros_acc)
    out = pl.pallas_call(
        _tc_out_body,
        out_shape=jax.ShapeDtypeStruct((N, D_OUT), f32),
    )(t3[0, :N], t3[1, :N], v3, p3, dinv, w3b, b3r)
    return out


# speedup vs baseline: 51.8163x; 1.5040x over previous
import functools

import jax
import jax.numpy as jnp
from jax import lax
from jax.experimental import pallas as pl
from jax.experimental.pallas import tpu as pltpu
from jax.experimental.pallas import tpu_sc as plsc

N = 10000
E = 320000
D_IN = 128
H = 16
D_OUT = 128

NC = 2
NS = 16
NW = NC * NS
CHUNK = 128
CHUNKS_PW = 80
EPW = CHUNK * CHUNKS_PW
E_PAD = EPW * NW
N_ACC = 10240
ROWS_PT = N_ACC // NS
NBUF = 4
BLKS = CHUNKS_PW // NBUF
N_Y = N // NS

NP = N // 8
NP_ACC = N_ACC // 8


def _sc_degree_body(row_hbm, ones_hbm, zeros_hbm, out_hbm,
                    rows_v, msg_v, acc_sh, sem):
    cid = lax.axis_index("c")
    sid = lax.axis_index("s")
    wid = cid * NS + sid
    pltpu.sync_copy(zeros_hbm.at[pl.ds(sid * ROWS_PT, ROWS_PT)],
                    acc_sh.at[pl.ds(sid * ROWS_PT, ROWS_PT)])
    pltpu.sync_copy(row_hbm.at[wid], rows_v)
    pltpu.sync_copy(ones_hbm, msg_v)
    plsc.subcore_barrier()

    def body(c, carry):
        pltpu.sync_copy(msg_v, acc_sh.at[rows_v.at[c]], add=True)
        return carry
    lax.fori_loop(0, CHUNKS_PW, body, 0)
    plsc.subcore_barrier()
    pltpu.sync_copy(acc_sh.at[pl.ds(sid * ROWS_PT, ROWS_PT)],
                    out_hbm.at[cid, pl.ds(sid * ROWS_PT, ROWS_PT)])


@functools.cache
def _sc_degree():
    mesh = plsc.VectorSubcoreMesh(
        core_axis_name="c", subcore_axis_name="s",
        num_cores=NC, num_subcores=NS)
    return pl.kernel(
        _sc_degree_body, mesh=mesh,
        out_type=jax.ShapeDtypeStruct((NC, N_ACC, H), jnp.float32),
        scratch_types=[
            pltpu.VMEM((CHUNKS_PW, CHUNK), jnp.int32),
            pltpu.VMEM((CHUNK, H), jnp.float32),
            pltpu.VMEM_SHARED((N_ACC, H), jnp.float32),
            pltpu.SemaphoreType.DMA,
        ],
        compiler_params=pltpu.CompilerParams(use_tc_tiling_on_sc=False),
        name="sc_degree",
    )


def _agg_loop(rows_v, cols_v, msg_v, acc_sh, y_sh, sems):
    for b in range(NBUF):
        pltpu.async_copy(y_sh.at[cols_v.at[b]], msg_v.at[b], sems[b])

    def blk(cb, carry):
        for b in range(NBUF):
            c = cb * NBUF + b
            pltpu.make_async_copy(y_sh.at[cols_v.at[c]],
                                  msg_v.at[b], sems[b]).wait()
            pltpu.sync_copy(msg_v.at[b], acc_sh.at[rows_v.at[c]], add=True)
            pltpu.async_copy(y_sh.at[cols_v.at[c + NBUF]],
                             msg_v.at[b], sems[b])
        return carry
    lax.fori_loop(0, BLKS - 1, blk, 0)
    for b in range(NBUF):
        c = (BLKS - 1) * NBUF + b
        pltpu.make_async_copy(y_sh.at[cols_v.at[c]],
                              msg_v.at[b], sems[b]).wait()
        pltpu.sync_copy(msg_v.at[b], acc_sh.at[rows_v.at[c]], add=True)


def _sc_agg_body(row_hbm, col_hbm, y_hbm, zeros_hbm, out_hbm,
                 rows_v, cols_v, msg_v, acc_sh, y_sh, sem0, sem1, sem2, sem3):
    sems = (sem0, sem1, sem2, sem3)
    cid = lax.axis_index("c")
    sid = lax.axis_index("s")
    wid = cid * NS + sid
    pltpu.sync_copy(zeros_hbm.at[pl.ds(sid * ROWS_PT, ROWS_PT)],
                    acc_sh.at[pl.ds(sid * ROWS_PT, ROWS_PT)])
    pltpu.sync_copy(y_hbm.at[pl.ds(sid * N_Y, N_Y)],
                    y_sh.at[pl.ds(sid * N_Y, N_Y)])
    pltpu.sync_copy(row_hbm.at[wid], rows_v)
    pltpu.sync_copy(col_hbm.at[wid], cols_v)
    plsc.subcore_barrier()
    _agg_loop(rows_v, cols_v, msg_v, acc_sh, y_sh, sems)
    plsc.subcore_barrier()
    pltpu.sync_copy(acc_sh.at[pl.ds(sid * ROWS_PT, ROWS_PT)],
                    out_hbm.at[cid, pl.ds(sid * ROWS_PT, ROWS_PT)])


def _sc_agg_scaled_body(row_hbm, col_hbm, z_hbm, dinv_hbm, zeros_hbm, out_hbm,
                        rows_v, cols_v, msg_v, acc_sh, y_sh, zb_v, db_v,
                        sem0, sem1, sem2, sem3):
    sems = (sem0, sem1, sem2, sem3)
    cid = lax.axis_index("c")
    sid = lax.axis_index("s")
    wid = cid * NS + sid
    pltpu.sync_copy(zeros_hbm.at[pl.ds(sid * ROWS_PT, ROWS_PT)],
                    acc_sh.at[pl.ds(sid * ROWS_PT, ROWS_PT)])
    pltpu.sync_copy(z_hbm.at[pl.ds(sid * N_Y, N_Y)], zb_v)
    pltpu.sync_copy(dinv_hbm.at[pl.ds(sid * N_Y, N_Y)], db_v)

    def scale(r, carry):
        zb_v[r] = zb_v[r] * db_v[r]
        return carry
    lax.fori_loop(0, N_Y, scale, 0)
    pltpu.sync_copy(zb_v, y_sh.at[pl.ds(sid * N_Y, N_Y)])
    pltpu.sync_copy(row_hbm.at[wid], rows_v)
    pltpu.sync_copy(col_hbm.at[wid], cols_v)
    plsc.subcore_barrier()
    _agg_loop(rows_v, cols_v, msg_v, acc_sh, y_sh, sems)
    plsc.subcore_barrier()
    pltpu.sync_copy(acc_sh.at[pl.ds(sid * ROWS_PT, ROWS_PT)],
                    out_hbm.at[cid, pl.ds(sid * ROWS_PT, ROWS_PT)])


_AGG_SCRATCH = [
    pltpu.VMEM((CHUNKS_PW, CHUNK), jnp.int32),
    pltpu.VMEM((CHUNKS_PW, CHUNK), jnp.int32),
    pltpu.VMEM((NBUF, CHUNK, H), jnp.float32),
    pltpu.VMEM_SHARED((N_ACC, H), jnp.float32),
    pltpu.VMEM_SHARED((N, H), jnp.float32),
]
_SEMS = [pltpu.SemaphoreType.DMA] * NBUF


@functools.cache
def _sc_agg():
    mesh = plsc.VectorSubcoreMesh(
        core_axis_name="c", subcore_axis_name="s",
        num_cores=NC, num_subcores=NS)
    return pl.kernel(
        _sc_agg_body, mesh=mesh,
        out_type=jax.ShapeDtypeStruct((NC, N_ACC, H), jnp.float32),
        scratch_types=_AGG_SCRATCH + _SEMS,
        compiler_params=pltpu.CompilerParams(use_tc_tiling_on_sc=False),
        name="sc_agg",
    )


@functools.cache
def _sc_agg_scaled():
    mesh = plsc.VectorSubcoreMesh(
        core_axis_name="c", subcore_axis_name="s",
        num_cores=NC, num_subcores=NS)
    return pl.kernel(
        _sc_agg_scaled_body, mesh=mesh,
        out_type=jax.ShapeDtypeStruct((NC, N_ACC, H), jnp.float32),
        scratch_types=_AGG_SCRATCH + [
            pltpu.VMEM((N_Y, H), jnp.float32),
            pltpu.VMEM((N_Y, H), jnp.float32),
        ] + _SEMS,
        compiler_params=pltpu.CompilerParams(use_tc_tiling_on_sc=False),
        name="sc_agg_scaled",
    )


def _fold8(v128):
    acc = v128[:, 0:H]
    for k in range(1, 8):
        acc = acc + v128[:, k * H:(k + 1) * H]
    return acc


def _tile8(v16):
    return jnp.concatenate([v16] * 8, axis=1)


def _bn_relu_packed(h, g, beta):
    mu = _tile8(_fold8(jnp.mean(h, axis=0, keepdims=True)) / 8.0)
    ex2 = _tile8(_fold8(jnp.mean(h * h, axis=0, keepdims=True)) / 8.0)
    var = ex2 - mu * mu
    return jnp.maximum((h - mu) / jnp.sqrt(var + 1e-5) * g + beta, 0.0)


def _tc1_body(degp, x, w1t, w1b, dinvp_o, zn_o, p1n_o):
    deg = degp[0, :NP] + degp[1, :NP] + 1.0
    dinvp_o[...] = lax.rsqrt(deg)
    zn_o[...] = jnp.dot(x[...], w1b[...], preferred_element_type=jnp.float32)
    p1n_o[...] = jnp.dot(x[...], w1t[...], preferred_element_type=jnp.float32)


def _tc2_body(s1p, dinvp, zp, p1p, b1t, g1t, bt1t, bdw2t, v2p_o, p2p_o):
    dinv = dinvp[...]
    u1 = dinv * zp[...]
    h = p1p[...] + dinv * (s1p[0, :NP] + s1p[1, :NP] + u1) + b1t[...]
    hn = _bn_relu_packed(h, g1t[...], bt1t[...])
    v2p_o[...] = dinv * hn
    p2p_o[...] = jnp.dot(hn, bdw2t[...], preferred_element_type=jnp.float32)


def _tc3_body(t2p, dinvp, v2p, p2p, b2t, g2t, bt2t, bdw2b, bdw3t,
              v3p_o, p3p_o):
    dinv = dinvp[...]
    agg2 = dinv * (t2p[0, :NP] + t2p[1, :NP] + v2p[...])
    h = (p2p[...] +
         jnp.dot(agg2, bdw2b[...], preferred_element_type=jnp.float32) +
         b2t[...])
    hn = _bn_relu_packed(h, g2t[...], bt2t[...])
    v3p_o[...] = dinv * hn
    p3p_o[...] = jnp.dot(hn, bdw3t[...], preferred_element_type=jnp.float32)


def _tc4_body(t3p, dinvp, v3p, p3p, bdw3b, b3t, outp_o):
    agg3 = dinvp[...] * (t3p[0, :NP] + t3p[1, :NP] + v3p[...])
    outp_o[...] = (p3p[...] +
                   jnp.dot(agg3, bdw3b[...], preferred_element_type=jnp.float32)
                   + b3t[...])


def _bd8(w):
    return jnp.kron(jnp.eye(8, dtype=w.dtype), w)


def kernel(edge_index, features, W1, b1, g1, beta1, W2, b2, g2, beta2, W3, b3):
    f32 = jnp.float32
    row = edge_index[0]
    col = edge_index[1]
    pad = E_PAD - E
    row_p = jnp.concatenate([row, jnp.full((pad,), N, jnp.int32)])
    col_p = jnp.concatenate([col, jnp.zeros((pad,), jnp.int32)])
    row_p = row_p.reshape(NW, CHUNKS_PW, CHUNK)
    col_p = col_p.reshape(NW, CHUNKS_PW, CHUNK)
    zeros_acc = jnp.zeros((N_ACC, H), f32)
    ones_msg = jnp.ones((CHUNK, H), f32)

    w1t, w1b = W1[:D_IN], W1[D_IN:]
    w2t, w2b = W2[:H], W2[H:]
    w3t, w3b = W3[:H], W3[H:]
    bdw2t, bdw2b = _bd8(w2t), _bd8(w2b)
    bdw3t, bdw3b = _bd8(w3t), _bd8(w3b)
    b3t = jnp.tile(b3, 8).reshape(1, 8 * D_OUT)
    b1t, g1t, bt1t = (jnp.tile(v, 8).reshape(1, 128) for v in (b1, g1, beta1))
    b2t, g2t, bt2t = (jnp.tile(v, 8).reshape(1, 128) for v in (b2, g2, beta2))

    deg2 = _sc_degree()(row_p, ones_msg, zeros_acc)
    deg2p = deg2.reshape(NC, NP_ACC, 128)

    dinvp, zn, p1n = pl.pallas_call(
        _tc1_body,
        out_shape=[jax.ShapeDtypeStruct((NP, 128), f32),
                   jax.ShapeDtypeStruct((N, H), f32),
                   jax.ShapeDtypeStruct((N, H), f32)],
    )(deg2p, features, w1t, w1b)

    zp = zn.reshape(NP, 128)
    p1p = p1n.reshape(NP, 128)
    zlin = zp.reshape(N, H)
    dinvlin = dinvp.reshape(N, H)

    s1 = _sc_agg_scaled()(row_p, col_p, zlin, dinvlin, zeros_acc)
    v2p, p2p = pl.pallas_call(
        _tc2_body,
        out_shape=[jax.ShapeDtypeStruct((NP, 128), f32)] * 2,
    )(s1.reshape(NC, NP_ACC, 128), dinvp, zp, p1p, b1t, g1t, bt1t, bdw2t)

    t2 = _sc_agg()(row_p, col_p, v2p.reshape(N, H), zeros_acc)
    v3p, p3p = pl.pallas_call(
        _tc3_body,
        out_shape=[jax.ShapeDtypeStruct((NP, 128), f32),
                   jax.ShapeDtypeStruct((NP, 8 * D_OUT), f32)],
    )(t2.reshape(NC, NP_ACC, 128), dinvp, v2p, p2p, b2t, g2t, bt2t,
      bdw2b, bdw3t)

    t3 = _sc_agg()(row_p, col_p, v3p.reshape(N, H), zeros_acc)
    outp = pl.pallas_call(
        _tc4_body,
        out_shape=jax.ShapeDtypeStruct((NP, 8 * D_OUT), f32),
    )(t3.reshape(NC, NP_ACC, 128), dinvp, v3p, p3p, bdw3b, b3t)
    return outp.reshape(N, D_OUT)
